# baseline (speedup 1.0000x reference)
import jax
import jax.numpy as jnp
from jax import lax
from jax.experimental import pallas as pl
from jax.experimental.pallas import tpu as pltpu
from jax.experimental.pallas import tpu_sc as plsc

NC = 2
NS = 16
L = 16
NW = NC * NS

E = 320000
N = 10000
D = 128
DP = D // 2
EPW = E // NW
CH = 192
PCH = 48
NFULL = EPW // CH
TAIL_OFF = EPW - CH
NCHUNK = NFULL + 1
NPAIR = NFULL // 2
SW = 129


def _body(x_hbm, src_hbm, dst_hbm, out_hbm,
          sidx_v, didx_v, out_v, sr0, sr1, dr0, dr1, tr_v, stg_v, x_spm,
          sem_s0, sem_d0, sem_s1, sem_d1):
  sid = lax.axis_index("s")
  wid = sid * NC + lax.axis_index("c")
  base = wid * EPW
  rows0 = lax.broadcasted_iota(jnp.int32, (L,), 0)

  def pack_rows(nrows, r0):
    pltpu.sync_copy(x_hbm.at[pl.ds(r0, nrows)], stg_v.at[pl.ds(0, nrows)])
    def row_body(r, carry):
      for k in range(DP // L):
        a = stg_v[r, pl.ds(2 * k * L, L)]
        b = stg_v[r, pl.ds((2 * k + 1) * L, L)]
        packed = plsc.bitcast(
            plsc.pack(a, b, format=plsc.PackFormat.INTERLEAVED), jnp.int32)
        sr0[r, pl.ds(k * L, L)] = packed
      return carry
    lax.fori_loop(0, nrows, row_body, 0, unroll=False)
    pltpu.sync_copy(sr0.at[pl.ds(0, nrows)], x_spm.at[pl.ds(r0, nrows)])

  def stage_body(c, carry):
    pack_rows(PCH, pl.multiple_of((sid + c * NS) * PCH, 8))
    return carry
  lax.fori_loop(0, 13, stage_body, 0, unroll=False)

  @pl.when(sid == 0)
  def _stage_tail():
    pack_rows(16, 9984)

  plsc.subcore_barrier()

  pltpu.sync_copy(src_hbm.at[pl.ds(base, EPW)], sidx_v)
  pltpu.sync_copy(dst_hbm.at[pl.ds(base, EPW)], didx_v)

  def fire(off, srows, drows, sem_s, sem_d):
    pltpu.async_copy(x_spm.at[sidx_v.at[pl.ds(off, CH)]], srows, sem_s)
    pltpu.async_copy(x_spm.at[didx_v.at[pl.ds(off, CH)]], drows, sem_d)

  def wait(srows, drows, sem_s, sem_d):
    pltpu.make_async_copy(x_spm.at[sidx_v.at[pl.ds(0, CH)]], srows, sem_s).wait()
    pltpu.make_async_copy(x_spm.at[didx_v.at[pl.ds(0, CH)]], drows, sem_d).wait()

  colbase = rows0 * SW

  def compute(off, srows, drows):
    def edge_body(e, carry):
      acc = jnp.zeros((L,), jnp.float32)
      for k in range(DP // L):
        s = plsc.bitcast(srows[e, pl.ds(k * L, L)], jnp.bfloat16)
        d = plsc.bitcast(drows[e, pl.ds(k * L, L)], jnp.bfloat16)
        prod = s * d
        pa, pb = plsc.unpack(prod, format=plsc.PackFormat.INTERLEAVED)
        acc = acc + pa + pb
      plsc.store_scatter(tr_v, [colbase + e], acc)
      return carry
    lax.fori_loop(0, CH, edge_body, 0, unroll=False)

    for cg in range(CH // L):
      tot = jnp.zeros((L,), jnp.float32)
      for k in range(L):
        tot = tot + plsc.load_gather(
            tr_v, [jnp.full((L,), k * SW + cg * L, jnp.int32) + rows0])
      out_v[pl.ds(off + cg * L, L)] = tot

  fire(0, sr0, dr0, sem_s0, sem_d0)

  def pair_body(t, carry):
    j0 = 2 * t
    fire((j0 + 1) * CH, sr1, dr1, sem_s1, sem_d1)
    wait(sr0, dr0, sem_s0, sem_d0)
    compute(j0 * CH, sr0, dr0)
    off2 = jnp.minimum((j0 + 2) * CH, TAIL_OFF)
    fire(off2, sr0, dr0, sem_s0, sem_d0)
    wait(sr1, dr1, sem_s1, sem_d1)
    compute((j0 + 1) * CH, sr1, dr1)
    return carry

  lax.fori_loop(0, NPAIR, pair_body, 0, unroll=False)

  if NFULL % 2 == 1:
    fire(TAIL_OFF, sr1, dr1, sem_s1, sem_d1)
    wait(sr0, dr0, sem_s0, sem_d0)
    compute((NFULL - 1) * CH, sr0, dr0)
    wait(sr1, dr1, sem_s1, sem_d1)
    compute(TAIL_OFF, sr1, dr1)
  else:
    wait(sr0, dr0, sem_s0, sem_d0)
    compute(TAIL_OFF, sr0, dr0)

  pltpu.sync_copy(out_v, out_hbm.at[pl.ds(base, EPW)])


@jax.jit
def kernel(x, edge_index):
  mesh = plsc.VectorSubcoreMesh(core_axis_name="c", subcore_axis_name="s")
  k = pl.kernel(
      _body,
      out_type=jax.ShapeDtypeStruct((E,), jnp.float32),
      mesh=mesh,
      compiler_params=pltpu.CompilerParams(
          needs_layout_passes=False, use_tc_tiling_on_sc=False),
      scratch_types=[
          pltpu.VMEM((EPW,), jnp.int32),
          pltpu.VMEM((EPW,), jnp.int32),
          pltpu.VMEM((EPW,), jnp.float32),
          pltpu.VMEM((CH, DP), jnp.int32),
          pltpu.VMEM((CH, DP), jnp.int32),
          pltpu.VMEM((CH, DP), jnp.int32),
          pltpu.VMEM((CH, DP), jnp.int32),
          pltpu.VMEM((L * SW,), jnp.float32),
          pltpu.VMEM((PCH, D), jnp.float32),
          pltpu.VMEM_SHARED((N, DP), jnp.int32),
          pltpu.SemaphoreType.DMA,
          pltpu.SemaphoreType.DMA,
          pltpu.SemaphoreType.DMA,
          pltpu.SemaphoreType.DMA,
      ],
  )
  return k(x, edge_index[0], edge_index[1])


# speedup vs baseline: 1.1107x; 1.1107x over previous
import jax
import jax.numpy as jnp
from jax import lax
from jax.experimental import pallas as pl
from jax.experimental.pallas import tpu as pltpu
from jax.experimental.pallas import tpu_sc as plsc

NC = 2
NS = 16
L = 16
NW = NC * NS

E = 320000
D = 128
EPW = E // NW
CH = 128
NFULL = EPW // CH
TAIL_OFF = EPW - CH
NCHUNK = NFULL + 1
NPAIR = NFULL // 2
SW = 129


def _body(x_hbm, src_hbm, dst_hbm, out_hbm,
          sidx_v, didx_v, out_v, rv0, rv1, tr_v,
          sem0, sem1):
  wid = lax.axis_index("s") * NC + lax.axis_index("c")
  base = wid * EPW
  rows0 = lax.broadcasted_iota(jnp.int32, (L,), 0)

  pltpu.sync_copy(src_hbm.at[pl.ds(base, EPW)], sidx_v)
  pltpu.sync_copy(dst_hbm.at[pl.ds(base, EPW)], didx_v)

  def fire(off, rv, sem):
    pltpu.async_copy(x_hbm.at[sidx_v.at[pl.ds(off, CH)]], rv.at[pl.ds(0, CH)], sem)
    pltpu.async_copy(x_hbm.at[didx_v.at[pl.ds(off, CH)]], rv.at[pl.ds(CH, CH)], sem)

  def wait(rv, sem):
    pltpu.make_async_copy(x_hbm.at[pl.ds(0, 2 * CH)], rv, sem).wait()

  colbase = rows0 * SW

  def compute(off, rv):
    def edge_body(e, carry):
      acc = jnp.zeros((L,), jnp.float32)
      for k in range(D // L):
        s = rv[e, pl.ds(k * L, L)]
        d = rv[CH + e, pl.ds(k * L, L)]
        acc = acc + s * d
      plsc.store_scatter(tr_v, [colbase + e], acc)
      return carry
    lax.fori_loop(0, CH, edge_body, 0, unroll=False)

    for cg in range(CH // L):
      tot = jnp.zeros((L,), jnp.float32)
      for k in range(L):
        tot = tot + plsc.load_gather(tr_v, [jnp.full((L,), k * SW + cg * L, jnp.int32) + rows0])
      out_v[pl.ds(off + cg * L, L)] = tot

  fire(0, rv0, sem0)

  def pair_body(t, carry):
    j0 = 2 * t
    fire((j0 + 1) * CH, rv1, sem1)
    wait(rv0, sem0)
    compute(j0 * CH, rv0)
    off2 = jnp.minimum((j0 + 2) * CH, TAIL_OFF)
    fire(off2, rv0, sem0)
    wait(rv1, sem1)
    compute((j0 + 1) * CH, rv1)
    return carry

  lax.fori_loop(0, NPAIR, pair_body, 0, unroll=False)

  wait(rv0, sem0)
  compute(TAIL_OFF, rv0)

  pltpu.sync_copy(out_v, out_hbm.at[pl.ds(base, EPW)])


@jax.jit
def kernel(x, edge_index):
  mesh = plsc.VectorSubcoreMesh(core_axis_name="c", subcore_axis_name="s")
  k = pl.kernel(
      _body,
      out_type=jax.ShapeDtypeStruct((E,), jnp.float32),
      mesh=mesh,
      compiler_params=pltpu.CompilerParams(needs_layout_passes=False),
      scratch_types=[
          pltpu.VMEM((EPW,), jnp.int32),
          pltpu.VMEM((EPW,), jnp.int32),
          pltpu.VMEM((EPW,), jnp.float32),
          pltpu.VMEM((2 * CH, D), jnp.float32),
          pltpu.VMEM((2 * CH, D), jnp.float32),
          pltpu.VMEM((L * SW,), jnp.float32),
          pltpu.SemaphoreType.DMA,
          pltpu.SemaphoreType.DMA,
      ],
  )
  return k(x, edge_index[0], edge_index[1])


# speedup vs baseline: 1.1343x; 1.0212x over previous
import jax
import jax.numpy as jnp
from jax import lax
from jax.experimental import pallas as pl
from jax.experimental.pallas import tpu as pltpu
from jax.experimental.pallas import tpu_sc as plsc

NC = 2
NS = 16
L = 16
NW = NC * NS

E = 320000
D = 128
EPW = E // NW
CH = 128
NFULL = EPW // CH
TAIL_OFF = EPW - CH
NCHUNK = NFULL + 1
NPAIR = NFULL // 2
SW = 129


def _body(x_hbm, src_hbm, dst_hbm, out_hbm,
          sidx_v, didx_v, out_v, sr0, sr1, dr0, dr1, tr_v,
          sem_s0, sem_d0, sem_s1, sem_d1):
  wid = lax.axis_index("s") * NC + lax.axis_index("c")
  base = wid * EPW
  rows0 = lax.broadcasted_iota(jnp.int32, (L,), 0)

  pltpu.sync_copy(src_hbm.at[pl.ds(base, EPW)], sidx_v)
  pltpu.sync_copy(dst_hbm.at[pl.ds(base, EPW)], didx_v)

  def fire(off, srows, drows, sem_s, sem_d):
    pltpu.async_copy(x_hbm.at[sidx_v.at[pl.ds(off, CH)]], srows, sem_s)
    pltpu.async_copy(x_hbm.at[didx_v.at[pl.ds(off, CH)]], drows, sem_d)

  def wait(srows, drows, sem_s, sem_d):
    pltpu.make_async_copy(x_hbm.at[sidx_v.at[pl.ds(0, CH)]], srows, sem_s).wait()
    pltpu.make_async_copy(x_hbm.at[didx_v.at[pl.ds(0, CH)]], drows, sem_d).wait()

  colbase = rows0 * SW

  def compute(off, srows, drows):
    def edge_body(e, carry):
      acc = jnp.zeros((L,), jnp.float32)
      for k in range(D // L):
        s = srows[e, pl.ds(k * L, L)]
        d = drows[e, pl.ds(k * L, L)]
        acc = acc + s * d
      plsc.store_scatter(tr_v, [colbase + e], acc)
      return carry
    lax.fori_loop(0, CH, edge_body, 0, unroll=False)

    for cg in range(CH // L):
      tot = jnp.zeros((L,), jnp.float32)
      for k in range(L):
        tot = tot + plsc.load_gather(tr_v, [jnp.full((L,), k * SW + cg * L, jnp.int32) + rows0])
      out_v[pl.ds(off + cg * L, L)] = tot

  fire(0, sr0, dr0, sem_s0, sem_d0)

  def pair_body(t, carry):
    j0 = 2 * t
    fire((j0 + 1) * CH, sr1, dr1, sem_s1, sem_d1)
    wait(sr0, dr0, sem_s0, sem_d0)
    compute(j0 * CH, sr0, dr0)
    off2 = jnp.minimum((j0 + 2) * CH, TAIL_OFF)
    fire(off2, sr0, dr0, sem_s0, sem_d0)
    wait(sr1, dr1, sem_s1, sem_d1)
    compute((j0 + 1) * CH, sr1, dr1)
    return carry

  lax.fori_loop(0, NPAIR, pair_body, 0, unroll=False)

  wait(sr0, dr0, sem_s0, sem_d0)
  compute(TAIL_OFF, sr0, dr0)

  pltpu.sync_copy(out_v, out_hbm.at[pl.ds(base, EPW)])


@jax.jit
def kernel(x, edge_index):
  mesh = plsc.VectorSubcoreMesh(core_axis_name="c", subcore_axis_name="s")
  k = pl.kernel(
      _body,
      out_type=jax.ShapeDtypeStruct((E,), jnp.float32),
      mesh=mesh,
      compiler_params=pltpu.CompilerParams(needs_layout_passes=False),
      scratch_types=[
          pltpu.VMEM((EPW,), jnp.int32),
          pltpu.VMEM((EPW,), jnp.int32),
          pltpu.VMEM((EPW,), jnp.float32),
          pltpu.VMEM((CH, D), jnp.float32),
          pltpu.VMEM((CH, D), jnp.float32),
          pltpu.VMEM((CH, D), jnp.float32),
          pltpu.VMEM((CH, D), jnp.float32),
          pltpu.VMEM((L * SW,), jnp.float32),
          pltpu.SemaphoreType.DMA,
          pltpu.SemaphoreType.DMA,
          pltpu.SemaphoreType.DMA,
          pltpu.SemaphoreType.DMA,
      ],
  )
  return k(x, edge_index[0], edge_index[1])


# speedup vs baseline: 1.1684x; 1.0301x over previous
import jax
import jax.numpy as jnp
from jax import lax
from jax.experimental import pallas as pl
from jax.experimental.pallas import tpu as pltpu
from jax.experimental.pallas import tpu_sc as plsc

NC = 2
NS = 16
L = 16
NW = NC * NS

E = 320000
D = 128
EPW = E // NW
CH = 112
NFULL = EPW // CH
TAIL_OFF = EPW - CH
NCHUNK = NFULL + 1
NPAIR = NFULL // 2
SW = 129


def _body(x_hbm, src_hbm, dst_hbm, out_hbm,
          sidx_v, didx_v, out_v, rv0, rv1, rv2, tr_v,
          sem0, sem1, sem2):
  wid = lax.axis_index("s") * NC + lax.axis_index("c")
  base = wid * EPW
  rows0 = lax.broadcasted_iota(jnp.int32, (L,), 0)

  pltpu.sync_copy(src_hbm.at[pl.ds(base, EPW)], sidx_v)
  pltpu.sync_copy(dst_hbm.at[pl.ds(base, EPW)], didx_v)

  def fire(off, rv, sem):
    pltpu.async_copy(x_hbm.at[sidx_v.at[pl.ds(off, CH)]], rv.at[pl.ds(0, CH)], sem)
    pltpu.async_copy(x_hbm.at[didx_v.at[pl.ds(off, CH)]], rv.at[pl.ds(CH, CH)], sem)

  def wait(rv, sem):
    pltpu.make_async_copy(x_hbm.at[pl.ds(0, 2 * CH)], rv, sem).wait()

  colbase = rows0 * SW

  def compute(off, rv):
    def edge_body(e, carry):
      acc = jnp.zeros((L,), jnp.float32)
      for k in range(D // L):
        s = rv[e, pl.ds(k * L, L)]
        d = rv[CH + e, pl.ds(k * L, L)]
        acc = acc + s * d
      plsc.store_scatter(tr_v, [colbase + e], acc)
      return carry
    lax.fori_loop(0, CH, edge_body, 0, unroll=False)

    for cg in range(CH // L):
      tot = jnp.zeros((L,), jnp.float32)
      for k in range(L):
        tot = tot + plsc.load_gather(tr_v, [jnp.full((L,), k * SW + cg * L, jnp.int32) + rows0])
      out_v[pl.ds(off + cg * L, L)] = tot

  bufs = ((rv0, sem0), (rv1, sem1), (rv2, sem2))

  def off(c):
    return jnp.minimum(c * CH, TAIL_OFF)

  for i in range(3):
    fire(off(i), *bufs[i])

  def ring_body(t, carry):
    for i in range(3):
      c = 3 * t + i

      @pl.when(c < NCHUNK)
      def _do(i=i, c=c):
        wait(*bufs[i])
        compute(off(c), bufs[i][0])

        @pl.when(c + 3 < NCHUNK)
        def _refire():
          fire(off(c + 3), *bufs[i])
    return carry

  lax.fori_loop(0, (NCHUNK + 2) // 3, ring_body, 0, unroll=False)

  pltpu.sync_copy(out_v, out_hbm.at[pl.ds(base, EPW)])


@jax.jit
def kernel(x, edge_index):
  mesh = plsc.VectorSubcoreMesh(core_axis_name="c", subcore_axis_name="s")
  k = pl.kernel(
      _body,
      out_type=jax.ShapeDtypeStruct((E,), jnp.float32),
      mesh=mesh,
      compiler_params=pltpu.CompilerParams(needs_layout_passes=False),
      scratch_types=[
          pltpu.VMEM((EPW,), jnp.int32),
          pltpu.VMEM((EPW,), jnp.int32),
          pltpu.VMEM((EPW,), jnp.float32),
          pltpu.VMEM((2 * CH, D), jnp.float32),
          pltpu.VMEM((2 * CH, D), jnp.float32),
          pltpu.VMEM((2 * CH, D), jnp.float32),
          pltpu.VMEM((L * SW,), jnp.float32),
          pltpu.SemaphoreType.DMA,
          pltpu.SemaphoreType.DMA,
          pltpu.SemaphoreType.DMA,
      ],
  )
  return k(x, edge_index[0], edge_index[1])
